# baseline (speedup 1.0000x reference)
import functools

import jax
import jax.numpy as jnp
from jax import lax
from jax.experimental import pallas as pl
from jax.experimental.pallas import tpu as pltpu
from jax.experimental.pallas import tpu_sc as plsc

_E = 8
_K = 2
_D = 768
_DFF = 2048
_T = 2048
_TK = _T * _K

_BT = 256
_NT = _TK // _BT + _E
_PMAX = _NT * _BT

_NC, _NS = 2, 16
_NW = _NC * _NS

_G_ROWS = _PMAX // _NW
_G_CH = 96
_C_ROWS = _T // _NW


def _gather_body(x_hbm, idx_hbm, out_hbm, idx_v, rows_v, sem):
    wid = lax.axis_index("s") * _NC + lax.axis_index("c")
    base = wid * _G_ROWS
    for c in range(_G_ROWS // _G_CH):
        b = base + c * _G_CH
        pltpu.sync_copy(idx_hbm.at[pl.ds(b, _G_CH)], idx_v)
        pltpu.async_copy(x_hbm.at[idx_v], rows_v, sem).wait()
        pltpu.sync_copy(rows_v, out_hbm.at[pl.ds(b, _G_CH)])


def _sc_gather(x, tok_for_pos):
    mesh = plsc.VectorSubcoreMesh(core_axis_name="c", subcore_axis_name="s")
    return pl.kernel(
        _gather_body,
        mesh=mesh,
        out_type=jax.ShapeDtypeStruct((_PMAX, _D), jnp.float32),
        scratch_types=[
            pltpu.VMEM((_G_CH,), jnp.int32),
            pltpu.VMEM((_G_CH, _D), jnp.float32),
            pltpu.SemaphoreType.DMA,
        ],
    )(x, tok_for_pos)


def _ffn_body(te_ref, va_ref, xs_ref, w_ref, g_ref, u_ref, d_ref, ys_ref):
    i = pl.program_id(0)

    @pl.when(va_ref[i] > 0)
    def _():
        x = xs_ref[...]
        g = g_ref[0]
        u = u_ref[0]
        d = d_ref[0]
        a = jnp.dot(x, g.T, preferred_element_type=jnp.float32)
        b = jnp.dot(x, u.T, preferred_element_type=jnp.float32)
        h = (a * jax.nn.sigmoid(a)) * b * w_ref[...]
        ys_ref[...] = jnp.dot(h, d.T, preferred_element_type=jnp.float32)


def _tc_ffn(te, valid, xs, wp, gate, up, down):
    grid_spec = pltpu.PrefetchScalarGridSpec(
        num_scalar_prefetch=2,
        grid=(_NT,),
        in_specs=[
            pl.BlockSpec((_BT, _D), lambda i, te, va: (i, 0)),
            pl.BlockSpec((_BT, 1), lambda i, te, va: (i, 0)),
            pl.BlockSpec((1, _DFF, _D), lambda i, te, va: (te[i], 0, 0)),
            pl.BlockSpec((1, _DFF, _D), lambda i, te, va: (te[i], 0, 0)),
            pl.BlockSpec((1, _D, _DFF), lambda i, te, va: (te[i], 0, 0)),
        ],
        out_specs=pl.BlockSpec((_BT, _D), lambda i, te, va: (i, 0)),
    )
    return pl.pallas_call(
        _ffn_body,
        grid_spec=grid_spec,
        out_shape=jax.ShapeDtypeStruct((_PMAX, _D), jnp.float32),
    )(te, valid, xs, wp, gate, up, down)


def _combine_body(ys_hbm, p0_hbm, p1_hbm, out_hbm, i0_v, i1_v, r0_v, r1_v,
                  s0, s1):
    wid = lax.axis_index("s") * _NC + lax.axis_index("c")
    b = wid * _C_ROWS
    pltpu.sync_copy(p0_hbm.at[pl.ds(b, _C_ROWS)], i0_v)
    pltpu.sync_copy(p1_hbm.at[pl.ds(b, _C_ROWS)], i1_v)
    cp0 = pltpu.async_copy(ys_hbm.at[i0_v], r0_v, s0)
    cp1 = pltpu.async_copy(ys_hbm.at[i1_v], r1_v, s1)
    cp0.wait()
    cp1.wait()

    def row_fn(r, carry):
        for cc in range(_D // 16):
            sl = pl.ds(cc * 16, 16)
            r0_v[r, sl] += r1_v[r, sl]
        return carry

    lax.fori_loop(0, _C_ROWS, row_fn, 0)
    pltpu.sync_copy(r0_v, out_hbm.at[pl.ds(b, _C_ROWS)])


def _sc_combine(ys, p0, p1):
    mesh = plsc.VectorSubcoreMesh(core_axis_name="c", subcore_axis_name="s")
    return pl.kernel(
        _combine_body,
        mesh=mesh,
        out_type=jax.ShapeDtypeStruct((_T, _D), jnp.float32),
        scratch_types=[
            pltpu.VMEM((_C_ROWS,), jnp.int32),
            pltpu.VMEM((_C_ROWS,), jnp.int32),
            pltpu.VMEM((_C_ROWS, _D), jnp.float32),
            pltpu.VMEM((_C_ROWS, _D), jnp.float32),
            pltpu.SemaphoreType.DMA,
            pltpu.SemaphoreType.DMA,
        ],
    )(ys, p0, p1)


def _route(ids_flat, w_flat):
    order = jnp.argsort(ids_flat, stable=True).astype(jnp.int32)
    sorted_e = ids_flat[order]
    counts = jnp.zeros((_E,), jnp.int32).at[ids_flat].add(1)
    offsets = (jnp.cumsum(counts) - counts).astype(jnp.int32)
    pcounts = ((counts + _BT - 1) // _BT) * _BT
    cum_p = jnp.cumsum(pcounts).astype(jnp.int32)
    poff = cum_p - pcounts
    ranks = jnp.arange(_TK, dtype=jnp.int32)
    pos = poff[sorted_e] + (ranks - offsets[sorted_e])
    tok_for_pos = jnp.zeros((_PMAX,), jnp.int32).at[pos].set(order // _K)
    w_for_pos = jnp.zeros((_PMAX,), jnp.float32).at[pos].set(w_flat[order])
    pos_of_slot = jnp.zeros((_TK,), jnp.int32).at[order].set(pos)
    pos2 = pos_of_slot.reshape(_T, _K)
    p0 = pos2[:, 0] + 0
    p1 = pos2[:, 1] + 0
    tile_starts = jnp.arange(_NT, dtype=jnp.int32) * _BT
    te = jnp.minimum(
        jnp.searchsorted(cum_p, tile_starts, side="right"), _E - 1
    ).astype(jnp.int32)
    valid = (tile_starts < cum_p[-1]).astype(jnp.int32)
    return tok_for_pos, w_for_pos, p0, p1, te, valid


def kernel(hidden_states, topk_ids, topk_weights, gate_proj, up_proj,
           down_proj):
    B, S, D = hidden_states.shape
    x = hidden_states.reshape(B * S, D)
    ids_flat = topk_ids.reshape(-1).astype(jnp.int32)
    w_flat = topk_weights.reshape(-1).astype(jnp.float32)

    tok_for_pos, w_for_pos, p0, p1, te, valid = _route(ids_flat, w_flat)

    xs = _sc_gather(x, tok_for_pos)
    ys = _tc_ffn(te, valid, xs, w_for_pos[:, None], gate_proj, up_proj,
                 down_proj)
    out = _sc_combine(ys, p0, p1)
    return out.reshape(B, S, D)


# speedup vs baseline: 1.8559x; 1.8559x over previous
import functools

import jax
import jax.numpy as jnp
from jax import lax
from jax.experimental import pallas as pl
from jax.experimental.pallas import tpu as pltpu
from jax.experimental.pallas import tpu_sc as plsc

_E = 8
_K = 2
_D = 768
_DFF = 2048
_T = 2048
_TK = _T * _K

_BT = 256
_NT = _TK // _BT + _E
_PMAX = _NT * _BT

_NC, _NS = 2, 16
_NW = _NC * _NS

_W_ROWS = _T // _NW


def _dispatch_body(x_hbm, p0_hbm, p1_hbm, xs_hbm, xv, i0_v, i1_v, s0, s1):
    wid = lax.axis_index("s") * _NC + lax.axis_index("c")
    b = wid * _W_ROWS
    pltpu.sync_copy(p0_hbm.at[pl.ds(b, _W_ROWS)], i0_v)
    pltpu.sync_copy(p1_hbm.at[pl.ds(b, _W_ROWS)], i1_v)
    pltpu.sync_copy(x_hbm.at[pl.ds(b, _W_ROWS)], xv)
    c0 = pltpu.async_copy(xv, xs_hbm.at[i0_v], s0)
    c1 = pltpu.async_copy(xv, xs_hbm.at[i1_v], s1)
    c0.wait()
    c1.wait()


def _sc_dispatch(x, p0, p1):
    mesh = plsc.VectorSubcoreMesh(core_axis_name="c", subcore_axis_name="s")
    return pl.kernel(
        _dispatch_body,
        mesh=mesh,
        out_type=jax.ShapeDtypeStruct((_PMAX, _D), jnp.float32),
        scratch_types=[
            pltpu.VMEM((_W_ROWS, _D), jnp.float32),
            pltpu.VMEM((_W_ROWS,), jnp.int32),
            pltpu.VMEM((_W_ROWS,), jnp.int32),
            pltpu.SemaphoreType.DMA,
            pltpu.SemaphoreType.DMA,
        ],
    )(x, p0, p1)


def _ffn_body(te_ref, va_ref, xs_ref, w_ref, g_ref, u_ref, d_ref, ys_ref):
    i = pl.program_id(0)

    @pl.when(va_ref[i] > 0)
    def _():
        x = xs_ref[...]
        g = g_ref[0]
        u = u_ref[0]
        d = d_ref[0]
        a = jnp.dot(x, g.T, preferred_element_type=jnp.float32)
        b = jnp.dot(x, u.T, preferred_element_type=jnp.float32)
        h = (a * jax.nn.sigmoid(a)) * b * w_ref[...]
        ys_ref[...] = jnp.dot(h, d.T, preferred_element_type=jnp.float32)


def _tc_ffn(te, valid, xs, wp, gate, up, down):
    grid_spec = pltpu.PrefetchScalarGridSpec(
        num_scalar_prefetch=2,
        grid=(_NT,),
        in_specs=[
            pl.BlockSpec((_BT, _D), lambda i, te, va: (i, 0)),
            pl.BlockSpec((_BT, 1), lambda i, te, va: (i, 0)),
            pl.BlockSpec((1, _DFF, _D), lambda i, te, va: (te[i], 0, 0)),
            pl.BlockSpec((1, _DFF, _D), lambda i, te, va: (te[i], 0, 0)),
            pl.BlockSpec((1, _D, _DFF), lambda i, te, va: (te[i], 0, 0)),
        ],
        out_specs=pl.BlockSpec((_BT, _D), lambda i, te, va: (i, 0)),
    )
    return pl.pallas_call(
        _ffn_body,
        grid_spec=grid_spec,
        out_shape=jax.ShapeDtypeStruct((_PMAX, _D), jnp.float32),
    )(te, valid, xs, wp, gate, up, down)


def _combine_body(ys_hbm, p0_hbm, p1_hbm, out_hbm, i0_v, i1_v, r0_v, r1_v,
                  s0, s1):
    wid = lax.axis_index("s") * _NC + lax.axis_index("c")
    b = wid * _W_ROWS
    pltpu.sync_copy(p0_hbm.at[pl.ds(b, _W_ROWS)], i0_v)
    pltpu.sync_copy(p1_hbm.at[pl.ds(b, _W_ROWS)], i1_v)
    cp0 = pltpu.async_copy(ys_hbm.at[i0_v], r0_v, s0)
    cp1 = pltpu.async_copy(ys_hbm.at[i1_v], r1_v, s1)
    cp0.wait()
    cp1.wait()

    def row_fn(r, carry):
        for cc in range(_D // 16):
            sl = pl.ds(cc * 16, 16)
            r0_v[r, sl] += r1_v[r, sl]
        return carry

    lax.fori_loop(0, _W_ROWS, row_fn, 0)
    pltpu.sync_copy(r0_v, out_hbm.at[pl.ds(b, _W_ROWS)])


def _sc_combine(ys, p0, p1):
    mesh = plsc.VectorSubcoreMesh(core_axis_name="c", subcore_axis_name="s")
    return pl.kernel(
        _combine_body,
        mesh=mesh,
        out_type=jax.ShapeDtypeStruct((_T, _D), jnp.float32),
        scratch_types=[
            pltpu.VMEM((_W_ROWS,), jnp.int32),
            pltpu.VMEM((_W_ROWS,), jnp.int32),
            pltpu.VMEM((_W_ROWS, _D), jnp.float32),
            pltpu.VMEM((_W_ROWS, _D), jnp.float32),
            pltpu.SemaphoreType.DMA,
            pltpu.SemaphoreType.DMA,
        ],
    )(ys, p0, p1)


def _route(ids_flat, w_flat):
    eye = jnp.arange(_E, dtype=jnp.int32)
    onehot = (ids_flat[:, None] == eye[None, :]).astype(jnp.int32)
    csum = jnp.cumsum(onehot, axis=0)
    counts = csum[-1]
    pcounts = ((counts + _BT - 1) // _BT) * _BT
    cum_p = jnp.cumsum(pcounts)
    poff = cum_p - pcounts
    pos = jnp.sum(onehot * (poff[None, :] + csum - 1), axis=1).astype(jnp.int32)
    w_for_pos = jnp.zeros((_PMAX,), jnp.float32).at[pos].set(w_flat)
    pos2 = pos.reshape(_T, _K)
    p0 = pos2[:, 0] + 0
    p1 = pos2[:, 1] + 0
    tile_starts = jnp.arange(_NT, dtype=jnp.int32) * _BT
    te = jnp.sum(
        (tile_starts[:, None] >= cum_p[None, :]).astype(jnp.int32), axis=1
    )
    te = jnp.minimum(te, _E - 1).astype(jnp.int32)
    valid = (tile_starts < cum_p[-1]).astype(jnp.int32)
    return w_for_pos, p0, p1, te, valid


def kernel(hidden_states, topk_ids, topk_weights, gate_proj, up_proj,
           down_proj):
    B, S, D = hidden_states.shape
    x = hidden_states.reshape(B * S, D)
    ids_flat = topk_ids.reshape(-1).astype(jnp.int32)
    w_flat = topk_weights.reshape(-1).astype(jnp.float32)

    w_for_pos, p0, p1, te, valid = _route(ids_flat, w_flat)

    xs = _sc_dispatch(x, p0, p1)
    ys = _tc_ffn(te, valid, xs, w_for_pos[:, None], gate_proj, up_proj,
                 down_proj)
    out = _sc_combine(ys, p0, p1)
    return out.reshape(B, S, D)


# speedup vs baseline: 1.8567x; 1.0004x over previous
import functools

import jax
import jax.numpy as jnp
from jax import lax
from jax.experimental import pallas as pl
from jax.experimental.pallas import tpu as pltpu
from jax.experimental.pallas import tpu_sc as plsc

_E = 8
_K = 2
_D = 768
_DFF = 2048
_T = 2048
_TK = _T * _K

_BT = 256
_NT = _TK // _BT + _E
_PMAX = _NT * _BT

_NC, _NS = 2, 16
_NW = _NC * _NS

_W_ROWS = _T // _NW


def _dispatch_body(x_hbm, p0_hbm, p1_hbm, xs_hbm, xv, i0_v, i1_v, s0, s1):
    wid = lax.axis_index("s") * _NC + lax.axis_index("c")
    b = wid * _W_ROWS
    pltpu.sync_copy(p0_hbm.at[pl.ds(b, _W_ROWS)], i0_v)
    pltpu.sync_copy(p1_hbm.at[pl.ds(b, _W_ROWS)], i1_v)
    pltpu.sync_copy(x_hbm.at[pl.ds(b, _W_ROWS)], xv)
    c0 = pltpu.async_copy(xv, xs_hbm.at[i0_v], s0)
    c1 = pltpu.async_copy(xv, xs_hbm.at[i1_v], s1)
    c0.wait()
    c1.wait()


def _sc_dispatch(x, p0, p1):
    mesh = plsc.VectorSubcoreMesh(core_axis_name="c", subcore_axis_name="s")
    return pl.kernel(
        _dispatch_body,
        mesh=mesh,
        out_type=jax.ShapeDtypeStruct((_PMAX, _D), jnp.float32),
        scratch_types=[
            pltpu.VMEM((_W_ROWS, _D), jnp.float32),
            pltpu.VMEM((_W_ROWS,), jnp.int32),
            pltpu.VMEM((_W_ROWS,), jnp.int32),
            pltpu.SemaphoreType.DMA,
            pltpu.SemaphoreType.DMA,
        ],
    )(x, p0, p1)


def _ffn_body(te_ref, va_ref, xs_ref, w_ref, g_ref, u_ref, d_ref, ys_ref):
    i = pl.program_id(0)

    @pl.when(va_ref[i] > 0)
    def _():
        x = xs_ref[...].astype(jnp.bfloat16)
        g = g_ref[0].astype(jnp.bfloat16)
        u = u_ref[0].astype(jnp.bfloat16)
        d = d_ref[0].astype(jnp.bfloat16)
        a = jnp.dot(x, g.T, preferred_element_type=jnp.float32)
        b = jnp.dot(x, u.T, preferred_element_type=jnp.float32)
        h = (a * jax.nn.sigmoid(a)) * b * w_ref[...]
        ys_ref[...] = jnp.dot(h.astype(jnp.bfloat16), d.T,
                              preferred_element_type=jnp.float32)


def _tc_ffn(te, valid, xs, wp, gate, up, down):
    grid_spec = pltpu.PrefetchScalarGridSpec(
        num_scalar_prefetch=2,
        grid=(_NT,),
        in_specs=[
            pl.BlockSpec((_BT, _D), lambda i, te, va: (i, 0)),
            pl.BlockSpec((_BT, 1), lambda i, te, va: (i, 0)),
            pl.BlockSpec((1, _DFF, _D), lambda i, te, va: (te[i], 0, 0)),
            pl.BlockSpec((1, _DFF, _D), lambda i, te, va: (te[i], 0, 0)),
            pl.BlockSpec((1, _D, _DFF), lambda i, te, va: (te[i], 0, 0)),
        ],
        out_specs=pl.BlockSpec((_BT, _D), lambda i, te, va: (i, 0)),
    )
    return pl.pallas_call(
        _ffn_body,
        grid_spec=grid_spec,
        out_shape=jax.ShapeDtypeStruct((_PMAX, _D), jnp.float32),
    )(te, valid, xs, wp, gate, up, down)


def _combine_body(ys_hbm, p0_hbm, p1_hbm, out_hbm, i0_v, i1_v, r0_v, r1_v,
                  s0, s1):
    wid = lax.axis_index("s") * _NC + lax.axis_index("c")
    b = wid * _W_ROWS
    pltpu.sync_copy(p0_hbm.at[pl.ds(b, _W_ROWS)], i0_v)
    pltpu.sync_copy(p1_hbm.at[pl.ds(b, _W_ROWS)], i1_v)
    cp0 = pltpu.async_copy(ys_hbm.at[i0_v], r0_v, s0)
    cp1 = pltpu.async_copy(ys_hbm.at[i1_v], r1_v, s1)
    cp0.wait()
    cp1.wait()

    def row_fn(r, carry):
        for cc in range(_D // 16):
            sl = pl.ds(cc * 16, 16)
            r0_v[r, sl] += r1_v[r, sl]
        return carry

    lax.fori_loop(0, _W_ROWS, row_fn, 0)
    pltpu.sync_copy(r0_v, out_hbm.at[pl.ds(b, _W_ROWS)])


def _sc_combine(ys, p0, p1):
    mesh = plsc.VectorSubcoreMesh(core_axis_name="c", subcore_axis_name="s")
    return pl.kernel(
        _combine_body,
        mesh=mesh,
        out_type=jax.ShapeDtypeStruct((_T, _D), jnp.float32),
        scratch_types=[
            pltpu.VMEM((_W_ROWS,), jnp.int32),
            pltpu.VMEM((_W_ROWS,), jnp.int32),
            pltpu.VMEM((_W_ROWS, _D), jnp.float32),
            pltpu.VMEM((_W_ROWS, _D), jnp.float32),
            pltpu.SemaphoreType.DMA,
            pltpu.SemaphoreType.DMA,
        ],
    )(ys, p0, p1)


def _route(ids_flat, w_flat):
    eye = jnp.arange(_E, dtype=jnp.int32)
    onehot = (ids_flat[:, None] == eye[None, :]).astype(jnp.int32)
    csum = jnp.cumsum(onehot, axis=0)
    counts = csum[-1]
    pcounts = ((counts + _BT - 1) // _BT) * _BT
    cum_p = jnp.cumsum(pcounts)
    poff = cum_p - pcounts
    pos = jnp.sum(onehot * (poff[None, :] + csum - 1), axis=1).astype(jnp.int32)
    w_for_pos = jnp.zeros((_PMAX,), jnp.float32).at[pos].set(w_flat)
    pos2 = pos.reshape(_T, _K)
    p0 = pos2[:, 0] + 0
    p1 = pos2[:, 1] + 0
    tile_starts = jnp.arange(_NT, dtype=jnp.int32) * _BT
    te = jnp.sum(
        (tile_starts[:, None] >= cum_p[None, :]).astype(jnp.int32), axis=1
    )
    te = jnp.minimum(te, _E - 1).astype(jnp.int32)
    valid = (tile_starts < cum_p[-1]).astype(jnp.int32)
    return w_for_pos, p0, p1, te, valid


def kernel(hidden_states, topk_ids, topk_weights, gate_proj, up_proj,
           down_proj):
    B, S, D = hidden_states.shape
    x = hidden_states.reshape(B * S, D)
    ids_flat = topk_ids.reshape(-1).astype(jnp.int32)
    w_flat = topk_weights.reshape(-1).astype(jnp.float32)

    w_for_pos, p0, p1, te, valid = _route(ids_flat, w_flat)

    xs = _sc_dispatch(x, p0, p1)
    ys = _tc_ffn(te, valid, xs, w_for_pos[:, None], gate_proj, up_proj,
                 down_proj)
    out = _sc_combine(ys, p0, p1)
    return out.reshape(B, S, D)
